# baseline (speedup 1.0000x reference)
import dataclasses
import functools

import jax
import jax.numpy as jnp
from jax import lax
from jax.experimental import pallas as pl
from jax.experimental.pallas import tpu as pltpu
from jax.experimental.pallas import tpu_sc as plsc

_BS = 32
_L = 16


@functools.partial(jax.jit, static_argnums=(1, 2, 3))
def _sc_pool(x2d, b, f, d):
    mesh = plsc.VectorSubcoreMesh(core_axis_name="core", subcore_axis_name="subcore")
    cp = pltpu.CompilerParams()
    if "needs_layout_passes" in pltpu.CompilerParams.__dataclass_fields__:
        cp = dataclasses.replace(cp, needs_layout_passes=False)

    @functools.partial(
        pl.kernel,
        out_type=jax.ShapeDtypeStruct((b // _L, _L), jnp.float32),
        mesh=mesh,
        compiler_params=cp,
    )
    def k(x_hbm, o_hbm):
        def body(x_vmem, o_vmem):
            lane = lax.iota(jnp.int32, _L)

            @pl.loop(0, _BS // _L)
            def per_group(g):
                def per_sample(j, res):
                    i = g * _L + j
                    acc = jnp.zeros((d,), jnp.float32)
                    acc2 = jnp.zeros((d,), jnp.float32)
                    for jf in range(f):
                        v = x_vmem[i, pl.ds(jf * d, d)]
                        acc = acc + v
                        acc2 = acc2 + v * v
                    r = jnp.sum(acc * acc - acc2) * 0.5
                    return jnp.where(lane == j, r, res)

                o_vmem[g, :] = lax.fori_loop(
                    0, _L, per_sample, jnp.zeros((_L,), jnp.float32)
                )

        pltpu.emit_pipeline(
            body,
            grid=(b // _BS,),
            in_specs=[pl.BlockSpec((_BS, f * d), lambda i: (i, 0))],
            out_specs=[pl.BlockSpec((_BS // _L, _L), lambda i: (i, 0))],
            core_axis_name=("core", "subcore"),
            dimension_semantics=(pltpu.PARALLEL,),
        )(x_hbm, o_hbm)

    return k(x2d)


def kernel(feature_emb):
    b, f, d = feature_emb.shape
    x2d = feature_emb.reshape(b, f * d)
    return _sc_pool(x2d, b, f, d).reshape(b, 1)
